# baseline (speedup 1.0000x reference)
import jax
import jax.numpy as jnp
from jax import lax
from jax.experimental import pallas as pl
from jax.experimental.pallas import tpu as pltpu
from jax.experimental.pallas import tpu_sc as plsc

NUM_CORES = 2
NUM_SUBCORES = 16
LANES = 16
NUM_WORKERS = NUM_CORES * NUM_SUBCORES

BATCH = 16384
HIST = 50
EMBED_DIM = 64
VOCAB = 1000000
DREGS = EMBED_DIM // LANES

PAIR = 2
IDXROW = PAIR * HIST
BAGS_PER_WORKER = BATCH // NUM_WORKERS
CHUNK = 16
K = CHUNK // PAIR
NUM_CHUNKS = BAGS_PER_WORKER // CHUNK
NBUF = 2

DENSE_ROWS = VOCAB // 2
RP_CHUNK = 200
RP_TROWS = RP_CHUNK * 2
RP_STEPS = DENSE_ROWS // RP_CHUNK


RP_BLOCK = 4000


def _repack_body(in_ref, out_ref):
    h = pl.program_id(1)

    @pl.when(h == 0)
    def _():
        out_ref[:, 0:EMBED_DIM] = in_ref[...]

    @pl.when(h == 1)
    def _():
        out_ref[:, EMBED_DIM:2 * EMBED_DIM] = in_ref[...]


def _lookup_body(bi_hbm, table_hbm, out_hbm, idx_v, rows_v, out_v, gsems):
    wid = lax.axis_index("s") * NUM_CORES + lax.axis_index("c")
    bag0 = wid * BAGS_PER_WORKER
    row0 = bag0 // PAIR
    scale = jnp.full((LANES,), 1.0 / HIST, dtype=jnp.float32)

    def start_chunk(g, slot):
        base = pl.multiple_of(row0 + g * K, 8)
        pltpu.sync_copy(bi_hbm.at[pl.ds(base, K)], idx_v.at[slot])
        for j in range(K):
            pltpu.async_copy(table_hbm.at[idx_v.at[slot, j]],
                             rows_v.at[slot, j], gsems.at[slot])

    def process_chunk(g, slot):
        for j in range(K):
            pltpu.make_async_copy(table_hbm.at[idx_v.at[slot, j]],
                                  rows_v.at[slot, j], gsems.at[slot]).wait()

        def pair_body(jr, carry):
            for half in range(PAIR):
                accs = [rows_v[slot, jr, half * HIST, pl.ds(r * LANES, LANES)]
                        for r in range(DREGS)]
                for j in range(1, HIST):
                    for r in range(DREGS):
                        accs[r] = accs[r] + rows_v[slot, jr, half * HIST + j,
                                                   pl.ds(r * LANES, LANES)]
                for r in range(DREGS):
                    out_v[jr * PAIR + half,
                          pl.ds(r * LANES, LANES)] = accs[r] * scale
            return carry

        lax.fori_loop(0, K, pair_body, 0, unroll=False)
        obase = pl.multiple_of(bag0 + g * CHUNK, 8)
        pltpu.sync_copy(out_v, out_hbm.at[pl.ds(obase, CHUNK)])

        @pl.when(g + NBUF < NUM_CHUNKS)
        def _():
            start_chunk(g + NBUF, slot)

    for b in range(NBUF):
        start_chunk(b, b)

    def outer(gg, carry):
        for b in range(NBUF):
            process_chunk(gg * NBUF + b, b)
        return carry

    lax.fori_loop(0, NUM_CHUNKS // NBUF, outer, 0, unroll=False)


@jax.jit
def _embedding_bag_mean(batch_input, table):
    mesh = plsc.VectorSubcoreMesh(core_axis_name="c", subcore_axis_name="s")

    nblk = DENSE_ROWS // RP_BLOCK
    repack = pl.pallas_call(
        _repack_body,
        grid=(nblk, 2),
        in_specs=[
            pl.BlockSpec((RP_BLOCK, EMBED_DIM), lambda i, h: (i + h * nblk, 0)),
        ],
        out_specs=pl.BlockSpec((RP_BLOCK, 2 * EMBED_DIM), lambda i, h: (i, 0)),
        out_shape=jax.ShapeDtypeStruct((DENSE_ROWS, 2 * EMBED_DIM),
                                       jnp.float32),
    )
    dense = repack(table)

    lookup = pl.kernel(
        _lookup_body,
        out_type=jax.ShapeDtypeStruct((BATCH, EMBED_DIM), jnp.float32),
        mesh=mesh,
        compiler_params=pltpu.CompilerParams(use_tc_tiling_on_sc=False),
        scratch_types=[
            pltpu.VMEM((NBUF, K, IDXROW), jnp.int32),
            pltpu.VMEM((NBUF, K, IDXROW, EMBED_DIM), jnp.float32),
            pltpu.VMEM((CHUNK, EMBED_DIM), jnp.float32),
            pltpu.SemaphoreType.DMA((NBUF,)),
        ],
    )
    remapped = jnp.where(batch_input < DENSE_ROWS, 2 * batch_input,
                         2 * batch_input - (VOCAB - 1))
    return lookup(remapped.reshape(BATCH // PAIR, PAIR * HIST),
                  dense.reshape(VOCAB, EMBED_DIM))


def kernel(batch_input, table):
    return _embedding_bag_mean(batch_input, table)


# speedup vs baseline: 1.0194x; 1.0194x over previous
import jax
import jax.numpy as jnp
from jax import lax
from jax.experimental import pallas as pl
from jax.experimental.pallas import tpu as pltpu
from jax.experimental.pallas import tpu_sc as plsc

NUM_CORES = 2
NUM_SUBCORES = 16
LANES = 16
NUM_WORKERS = NUM_CORES * NUM_SUBCORES

BATCH = 16384
HIST = 50
EMBED_DIM = 64
VOCAB = 1000000
DREGS = EMBED_DIM // LANES

PAIR = 2
IDXROW = PAIR * HIST
BAGS_PER_WORKER = BATCH // NUM_WORKERS
CHUNK = 16
K = CHUNK // PAIR
NUM_CHUNKS = BAGS_PER_WORKER // CHUNK
NBUF = 2

DENSE_ROWS = VOCAB // 2
RP_CHUNK = 200
RP_TROWS = RP_CHUNK * 2
RP_STEPS = DENSE_ROWS // RP_CHUNK


RP_BC = 512
RP_GRID = 977
DENSE_PAD_ROWS = RP_GRID * RP_BC


def _repack_body(a_ref, b_ref, out_ref):
    out_ref[:, 0:EMBED_DIM] = a_ref[...].T
    out_ref[:, EMBED_DIM:2 * EMBED_DIM] = b_ref[...].T


def _lookup_body(bi_hbm, table_hbm, out_hbm, idx_v, rows_v, out_v, gsems):
    wid = lax.axis_index("s") * NUM_CORES + lax.axis_index("c")
    bag0 = wid * BAGS_PER_WORKER
    row0 = bag0 // PAIR
    scale = jnp.full((LANES,), 1.0 / HIST, dtype=jnp.float32)

    def start_chunk(g, slot):
        base = pl.multiple_of(row0 + g * K, 8)
        pltpu.sync_copy(bi_hbm.at[pl.ds(base, K)], idx_v.at[slot])
        for j in range(K):
            pltpu.async_copy(table_hbm.at[idx_v.at[slot, j]],
                             rows_v.at[slot, j], gsems.at[slot])

    def process_chunk(g, slot):
        for j in range(K):
            pltpu.make_async_copy(table_hbm.at[idx_v.at[slot, j]],
                                  rows_v.at[slot, j], gsems.at[slot]).wait()

        def pair_body(jr, carry):
            for half in range(PAIR):
                accs = [rows_v[slot, jr, half * HIST, pl.ds(r * LANES, LANES)]
                        for r in range(DREGS)]
                for j in range(1, HIST):
                    for r in range(DREGS):
                        accs[r] = accs[r] + rows_v[slot, jr, half * HIST + j,
                                                   pl.ds(r * LANES, LANES)]
                for r in range(DREGS):
                    out_v[jr * PAIR + half,
                          pl.ds(r * LANES, LANES)] = accs[r] * scale
            return carry

        lax.fori_loop(0, K, pair_body, 0, unroll=False)
        obase = pl.multiple_of(bag0 + g * CHUNK, 8)
        pltpu.sync_copy(out_v, out_hbm.at[pl.ds(obase, CHUNK)])

        @pl.when(g + NBUF < NUM_CHUNKS)
        def _():
            start_chunk(g + NBUF, slot)

    for b in range(NBUF):
        start_chunk(b, b)

    def outer(gg, carry):
        for b in range(NBUF):
            process_chunk(gg * NBUF + b, b)
        return carry

    lax.fori_loop(0, NUM_CHUNKS // NBUF, outer, 0, unroll=False)


@jax.jit
def _embedding_bag_mean(batch_input, table):
    mesh = plsc.VectorSubcoreMesh(core_axis_name="c", subcore_axis_name="s")

    table_t = table.T
    repack = pl.pallas_call(
        _repack_body,
        grid=(RP_GRID,),
        in_specs=[
            pl.BlockSpec((EMBED_DIM, RP_BC), lambda i: (0, 2 * i)),
            pl.BlockSpec((EMBED_DIM, RP_BC), lambda i: (0, 2 * i + 1)),
        ],
        out_specs=pl.BlockSpec((RP_BC, 2 * EMBED_DIM), lambda i: (i, 0)),
        out_shape=jax.ShapeDtypeStruct((DENSE_PAD_ROWS, 2 * EMBED_DIM),
                                       jnp.float32),
    )
    dense = repack(table_t, table_t)

    lookup = pl.kernel(
        _lookup_body,
        out_type=jax.ShapeDtypeStruct((BATCH, EMBED_DIM), jnp.float32),
        mesh=mesh,
        compiler_params=pltpu.CompilerParams(use_tc_tiling_on_sc=False),
        scratch_types=[
            pltpu.VMEM((NBUF, K, IDXROW), jnp.int32),
            pltpu.VMEM((NBUF, K, IDXROW, EMBED_DIM), jnp.float32),
            pltpu.VMEM((CHUNK, EMBED_DIM), jnp.float32),
            pltpu.SemaphoreType.DMA((NBUF,)),
        ],
    )
    bi = batch_input
    m = bi // (2 * RP_BC)
    rem = bi % (2 * RP_BC)
    h = rem // RP_BC
    r = rem % RP_BC
    remapped = 2 * (m * RP_BC + r) + h
    return lookup(remapped.reshape(BATCH // PAIR, PAIR * HIST),
                  dense.reshape(2 * DENSE_PAD_ROWS, EMBED_DIM))


def kernel(batch_input, table):
    return _embedding_bag_mean(batch_input, table)


# speedup vs baseline: 1.7720x; 1.7383x over previous
import jax
import jax.numpy as jnp
from jax import lax
from jax.experimental import pallas as pl
from jax.experimental.pallas import tpu as pltpu
from jax.experimental.pallas import tpu_sc as plsc

NUM_CORES = 2
NUM_SUBCORES = 16
LANES = 16
NUM_WORKERS = NUM_CORES * NUM_SUBCORES

BATCH = 16384
HIST = 50
EMBED_DIM = 64
VOCAB = 1000000
DREGS = EMBED_DIM // LANES

PAIR = 2
IDXROW = PAIR * HIST
BAGS_PER_WORKER = BATCH // NUM_WORKERS
CHUNK = 16
K = CHUNK // PAIR
NUM_CHUNKS = BAGS_PER_WORKER // CHUNK
NBUF = 2

DENSE_ROWS = VOCAB // 2
RP_CHUNK = 200
RP_TROWS = RP_CHUNK * 2
RP_STEPS = DENSE_ROWS // RP_CHUNK


RP_BC = 512
RP_IN = 4096
RP_GRID = 245
DENSE_PAD_ROWS = RP_GRID * RP_IN // 2


def _repack_body(in_ref, out_ref):
    y = in_ref[...].T
    for a in range(RP_IN // (2 * RP_BC)):
        out_ref[512 * a:512 * (a + 1), 0:EMBED_DIM] = (
            y[1024 * a:1024 * a + 512])
        out_ref[512 * a:512 * (a + 1), EMBED_DIM:2 * EMBED_DIM] = (
            y[1024 * a + 512:1024 * (a + 1)])


def _lookup_body(bi_hbm, table_hbm, out_hbm, idx_v, rows_v, out_v, gsems):
    wid = lax.axis_index("s") * NUM_CORES + lax.axis_index("c")
    bag0 = wid * BAGS_PER_WORKER
    row0 = bag0 // PAIR
    scale = jnp.full((LANES,), 1.0 / HIST, dtype=jnp.float32)

    def start_chunk(g, slot):
        base = pl.multiple_of(row0 + g * K, 8)
        pltpu.sync_copy(bi_hbm.at[pl.ds(base, K)], idx_v.at[slot])
        for j in range(K):
            pltpu.async_copy(table_hbm.at[idx_v.at[slot, j]],
                             rows_v.at[slot, j], gsems.at[slot])

    def process_chunk(g, slot):
        for j in range(K):
            pltpu.make_async_copy(table_hbm.at[idx_v.at[slot, j]],
                                  rows_v.at[slot, j], gsems.at[slot]).wait()

        def pair_body(jr, carry):
            for half in range(PAIR):
                accs = [rows_v[slot, jr, half * HIST, pl.ds(r * LANES, LANES)]
                        for r in range(DREGS)]
                for j in range(1, HIST):
                    for r in range(DREGS):
                        accs[r] = accs[r] + rows_v[slot, jr, half * HIST + j,
                                                   pl.ds(r * LANES, LANES)]
                for r in range(DREGS):
                    out_v[jr * PAIR + half,
                          pl.ds(r * LANES, LANES)] = accs[r] * scale
            return carry

        lax.fori_loop(0, K, pair_body, 0, unroll=False)
        obase = pl.multiple_of(bag0 + g * CHUNK, 8)
        pltpu.sync_copy(out_v, out_hbm.at[pl.ds(obase, CHUNK)])

        @pl.when(g + NBUF < NUM_CHUNKS)
        def _():
            start_chunk(g + NBUF, slot)

    for b in range(NBUF):
        start_chunk(b, b)

    def outer(gg, carry):
        for b in range(NBUF):
            process_chunk(gg * NBUF + b, b)
        return carry

    lax.fori_loop(0, NUM_CHUNKS // NBUF, outer, 0, unroll=False)


@jax.jit
def _embedding_bag_mean(batch_input, table):
    mesh = plsc.VectorSubcoreMesh(core_axis_name="c", subcore_axis_name="s")

    table_t = table.T
    repack = pl.pallas_call(
        _repack_body,
        grid=(RP_GRID,),
        in_specs=[
            pl.BlockSpec((EMBED_DIM, RP_IN), lambda i: (0, i)),
        ],
        out_specs=pl.BlockSpec((RP_IN // 2, 2 * EMBED_DIM), lambda i: (i, 0)),
        out_shape=jax.ShapeDtypeStruct((DENSE_PAD_ROWS, 2 * EMBED_DIM),
                                       jnp.float32),
    )
    dense = repack(table_t)

    lookup = pl.kernel(
        _lookup_body,
        out_type=jax.ShapeDtypeStruct((BATCH, EMBED_DIM), jnp.float32),
        mesh=mesh,
        compiler_params=pltpu.CompilerParams(use_tc_tiling_on_sc=False),
        scratch_types=[
            pltpu.VMEM((NBUF, K, IDXROW), jnp.int32),
            pltpu.VMEM((NBUF, K, IDXROW, EMBED_DIM), jnp.float32),
            pltpu.VMEM((CHUNK, EMBED_DIM), jnp.float32),
            pltpu.SemaphoreType.DMA((NBUF,)),
        ],
    )
    bi = batch_input
    m = bi // (2 * RP_BC)
    rem = bi % (2 * RP_BC)
    h = rem // RP_BC
    r = rem % RP_BC
    remapped = 2 * (m * RP_BC + r) + h
    return lookup(remapped.reshape(BATCH // PAIR, PAIR * HIST),
                  dense.reshape(2 * DENSE_PAD_ROWS, EMBED_DIM))


def kernel(batch_input, table):
    return _embedding_bag_mean(batch_input, table)


# speedup vs baseline: 2.2218x; 1.2538x over previous
import jax
import jax.numpy as jnp
from jax import lax
from jax.experimental import pallas as pl
from jax.experimental.pallas import tpu as pltpu
from jax.experimental.pallas import tpu_sc as plsc

NUM_CORES = 2
NUM_SUBCORES = 16
LANES = 16
NUM_WORKERS = NUM_CORES * NUM_SUBCORES

BATCH = 16384
HIST = 50
EMBED_DIM = 64
VOCAB = 1000000
DREGS = EMBED_DIM // LANES

PAIR = 2
IDXROW = PAIR * HIST
BAGS_PER_WORKER = BATCH // NUM_WORKERS
CHUNK = 16
K = CHUNK // PAIR
NUM_CHUNKS = BAGS_PER_WORKER // CHUNK
NBUF = 2

DENSE_ROWS = VOCAB // 2
RP_CHUNK = 200
RP_TROWS = RP_CHUNK * 2
RP_STEPS = DENSE_ROWS // RP_CHUNK


RP_BC = 512
RP_IN = 16384
RP_GRID = 62
DENSE_PAD_ROWS = RP_GRID * RP_IN // 2


def _repack_body(in_ref, out_ref):
    y = in_ref[...].T
    for a in range(RP_IN // (2 * RP_BC)):
        out_ref[512 * a:512 * (a + 1), 0:EMBED_DIM] = (
            y[1024 * a:1024 * a + 512])
        out_ref[512 * a:512 * (a + 1), EMBED_DIM:2 * EMBED_DIM] = (
            y[1024 * a + 512:1024 * (a + 1)])


def _lookup_body(bi_hbm, table_hbm, out_hbm, idx_v, rows_v, out_v, gsems):
    wid = lax.axis_index("s") * NUM_CORES + lax.axis_index("c")
    bag0 = wid * BAGS_PER_WORKER
    row0 = bag0 // PAIR
    scale = jnp.full((LANES,), 1.0 / HIST, dtype=jnp.float32)

    def start_chunk(g, slot):
        base = pl.multiple_of(row0 + g * K, 8)
        pltpu.sync_copy(bi_hbm.at[pl.ds(base, K)], idx_v.at[slot])
        for j in range(K):
            pltpu.async_copy(table_hbm.at[idx_v.at[slot, j]],
                             rows_v.at[slot, j], gsems.at[slot])

    def process_chunk(g, slot):
        for j in range(K):
            pltpu.make_async_copy(table_hbm.at[idx_v.at[slot, j]],
                                  rows_v.at[slot, j], gsems.at[slot]).wait()

        def pair_body(jr, carry):
            for half in range(PAIR):
                accs = [rows_v[slot, jr, half * HIST, pl.ds(r * LANES, LANES)]
                        for r in range(DREGS)]
                for j in range(1, HIST):
                    for r in range(DREGS):
                        accs[r] = accs[r] + rows_v[slot, jr, half * HIST + j,
                                                   pl.ds(r * LANES, LANES)]
                for r in range(DREGS):
                    out_v[jr * PAIR + half,
                          pl.ds(r * LANES, LANES)] = accs[r] * scale
            return carry

        lax.fori_loop(0, K, pair_body, 0, unroll=False)
        obase = pl.multiple_of(bag0 + g * CHUNK, 8)
        pltpu.sync_copy(out_v, out_hbm.at[pl.ds(obase, CHUNK)])

        @pl.when(g + NBUF < NUM_CHUNKS)
        def _():
            start_chunk(g + NBUF, slot)

    for b in range(NBUF):
        start_chunk(b, b)

    def outer(gg, carry):
        for b in range(NBUF):
            process_chunk(gg * NBUF + b, b)
        return carry

    lax.fori_loop(0, NUM_CHUNKS // NBUF, outer, 0, unroll=False)


@jax.jit
def _embedding_bag_mean(batch_input, table):
    mesh = plsc.VectorSubcoreMesh(core_axis_name="c", subcore_axis_name="s")

    table_t = table.T
    repack = pl.pallas_call(
        _repack_body,
        grid=(RP_GRID,),
        in_specs=[
            pl.BlockSpec((EMBED_DIM, RP_IN), lambda i: (0, i)),
        ],
        out_specs=pl.BlockSpec((RP_IN // 2, 2 * EMBED_DIM), lambda i: (i, 0)),
        out_shape=jax.ShapeDtypeStruct((DENSE_PAD_ROWS, 2 * EMBED_DIM),
                                       jnp.float32),
    )
    dense = repack(table_t)

    lookup = pl.kernel(
        _lookup_body,
        out_type=jax.ShapeDtypeStruct((BATCH, EMBED_DIM), jnp.float32),
        mesh=mesh,
        compiler_params=pltpu.CompilerParams(use_tc_tiling_on_sc=False),
        scratch_types=[
            pltpu.VMEM((NBUF, K, IDXROW), jnp.int32),
            pltpu.VMEM((NBUF, K, IDXROW, EMBED_DIM), jnp.float32),
            pltpu.VMEM((CHUNK, EMBED_DIM), jnp.float32),
            pltpu.SemaphoreType.DMA((NBUF,)),
        ],
    )
    bi = batch_input
    m = bi // (2 * RP_BC)
    rem = bi % (2 * RP_BC)
    h = rem // RP_BC
    r = rem % RP_BC
    remapped = 2 * (m * RP_BC + r) + h
    return lookup(remapped.reshape(BATCH // PAIR, PAIR * HIST),
                  dense.reshape(2 * DENSE_PAD_ROWS, EMBED_DIM))


def kernel(batch_input, table):
    return _embedding_bag_mean(batch_input, table)


# speedup vs baseline: 2.3055x; 1.0377x over previous
import jax
import jax.numpy as jnp
from jax import lax
from jax.experimental import pallas as pl
from jax.experimental.pallas import tpu as pltpu
from jax.experimental.pallas import tpu_sc as plsc

NUM_CORES = 2
NUM_SUBCORES = 16
LANES = 16
NUM_WORKERS = NUM_CORES * NUM_SUBCORES

BATCH = 16384
HIST = 50
EMBED_DIM = 64
VOCAB = 1000000
DREGS = EMBED_DIM // LANES

PAIR = 2
IDXROW = PAIR * HIST
BAGS_PER_WORKER = BATCH // NUM_WORKERS
CHUNK = 16
K = CHUNK // PAIR
NUM_CHUNKS = BAGS_PER_WORKER // CHUNK
NBUF = 2

DENSE_ROWS = VOCAB // 2
RP_CHUNK = 200
RP_TROWS = RP_CHUNK * 2
RP_STEPS = DENSE_ROWS // RP_CHUNK


RP_BC = 512
RP_IN = 32768
RP_GRID = 31
DENSE_PAD_ROWS = RP_GRID * RP_IN // 2


def _repack_body(in_ref, out_ref):
    y = in_ref[...].T
    for a in range(RP_IN // (2 * RP_BC)):
        out_ref[512 * a:512 * (a + 1), 0:EMBED_DIM] = (
            y[1024 * a:1024 * a + 512])
        out_ref[512 * a:512 * (a + 1), EMBED_DIM:2 * EMBED_DIM] = (
            y[1024 * a + 512:1024 * (a + 1)])


def _lookup_body(bi_hbm, table_hbm, out_hbm, idx_v, rows_v, out_v, gsems):
    wid = lax.axis_index("s") * NUM_CORES + lax.axis_index("c")
    bag0 = wid * BAGS_PER_WORKER
    row0 = bag0 // PAIR
    scale = jnp.full((LANES,), 1.0 / HIST, dtype=jnp.float32)

    def start_chunk(g, slot):
        base = pl.multiple_of(row0 + g * K, 8)
        pltpu.sync_copy(bi_hbm.at[pl.ds(base, K)], idx_v.at[slot])
        for j in range(K):
            pltpu.async_copy(table_hbm.at[idx_v.at[slot, j]],
                             rows_v.at[slot, j], gsems.at[slot])

    def process_chunk(g, slot):
        for j in range(K):
            pltpu.make_async_copy(table_hbm.at[idx_v.at[slot, j]],
                                  rows_v.at[slot, j], gsems.at[slot]).wait()

        def pair_body(jr, carry):
            for half in range(PAIR):
                accs = [rows_v[slot, jr, half * HIST, pl.ds(r * LANES, LANES)]
                        for r in range(DREGS)]
                for j in range(1, HIST):
                    for r in range(DREGS):
                        accs[r] = accs[r] + rows_v[slot, jr, half * HIST + j,
                                                   pl.ds(r * LANES, LANES)]
                for r in range(DREGS):
                    out_v[jr * PAIR + half,
                          pl.ds(r * LANES, LANES)] = accs[r] * scale
            return carry

        lax.fori_loop(0, K, pair_body, 0, unroll=False)
        obase = pl.multiple_of(bag0 + g * CHUNK, 8)
        pltpu.sync_copy(out_v, out_hbm.at[pl.ds(obase, CHUNK)])

        @pl.when(g + NBUF < NUM_CHUNKS)
        def _():
            start_chunk(g + NBUF, slot)

    for b in range(NBUF):
        start_chunk(b, b)

    def outer(gg, carry):
        for b in range(NBUF):
            process_chunk(gg * NBUF + b, b)
        return carry

    lax.fori_loop(0, NUM_CHUNKS // NBUF, outer, 0, unroll=False)


@jax.jit
def _embedding_bag_mean(batch_input, table):
    mesh = plsc.VectorSubcoreMesh(core_axis_name="c", subcore_axis_name="s")

    table_t = table.T
    repack = pl.pallas_call(
        _repack_body,
        grid=(RP_GRID,),
        in_specs=[
            pl.BlockSpec((EMBED_DIM, RP_IN), lambda i: (0, i)),
        ],
        out_specs=pl.BlockSpec((RP_IN // 2, 2 * EMBED_DIM), lambda i: (i, 0)),
        out_shape=jax.ShapeDtypeStruct((DENSE_PAD_ROWS, 2 * EMBED_DIM),
                                       jnp.float32),
    )
    dense = repack(table_t)

    lookup = pl.kernel(
        _lookup_body,
        out_type=jax.ShapeDtypeStruct((BATCH, EMBED_DIM), jnp.float32),
        mesh=mesh,
        compiler_params=pltpu.CompilerParams(use_tc_tiling_on_sc=False),
        scratch_types=[
            pltpu.VMEM((NBUF, K, IDXROW), jnp.int32),
            pltpu.VMEM((NBUF, K, IDXROW, EMBED_DIM), jnp.float32),
            pltpu.VMEM((CHUNK, EMBED_DIM), jnp.float32),
            pltpu.SemaphoreType.DMA((NBUF,)),
        ],
    )
    bi = batch_input
    m = bi // (2 * RP_BC)
    rem = bi % (2 * RP_BC)
    h = rem // RP_BC
    r = rem % RP_BC
    remapped = 2 * (m * RP_BC + r) + h
    return lookup(remapped.reshape(BATCH // PAIR, PAIR * HIST),
                  dense.reshape(2 * DENSE_PAD_ROWS, EMBED_DIM))


def kernel(batch_input, table):
    return _embedding_bag_mean(batch_input, table)
